# baseline (speedup 1.0000x reference)
import functools

import jax
import jax.numpy as jnp
from jax import lax
from jax.experimental import pallas as pl
from jax.experimental.pallas import tpu as pltpu
from jax.experimental.pallas import tpu_sc as plsc

D = 128
NW = 32
GATHER = 128
IDX_ROWS = 1
CHUNK = IDX_ROWS * GATHER
NBUF = 3
HBM_EVERY = 8
TABLE_ROWS = 8192


def _build(total_rows: int):
    per_w = total_rows // NW
    n_chunks = per_w // CHUNK
    n_outer = n_chunks // NBUF
    tail = [(c, c % NBUF) for c in range(n_outer * NBUF, n_chunks)]
    mesh = plsc.VectorSubcoreMesh(core_axis_name="c", subcore_axis_name="s")

    @functools.partial(
        pl.kernel,
        mesh=mesh,
        out_type=jax.ShapeDtypeStruct((total_rows, D), jnp.float32),
        scratch_types=[
            pltpu.VMEM((NBUF, IDX_ROWS, GATHER), jnp.int32),
            pltpu.VMEM((NBUF, CHUNK, D), jnp.float32),
            pltpu.VMEM_SHARED((TABLE_ROWS, D), jnp.float32),
            pltpu.SemaphoreType.DMA,
            pltpu.SemaphoreType.DMA,
        ] + [pltpu.SemaphoreType.DMA] * NBUF,
    )
    def gather_kernel(table_hbm, idx_hbm, out_hbm, idx_v, rows_v, table_sh,
                      isem, gsem, *osems):
        wid = lax.axis_index("s") * 2 + lax.axis_index("c")
        base_irow = wid * (per_w // GATHER)

        def idx_rows_of(c):
            return pl.ds(base_irow + c * IDX_ROWS, IDX_ROWS)

        def out_rows_of(c):
            return pl.ds((base_irow + c * IDX_ROWS) * GATHER, CHUNK)

        def _fire_from(table_ref, b):
            for j in range(IDX_ROWS):
                pltpu.async_copy(
                    table_ref.at[idx_v.at[b, j]],
                    rows_v.at[b, pl.ds(j * GATHER, GATHER)],
                    gsem,
                )

        def fire_gathers(b, c):
            pred = c % HBM_EVERY == 0
            if isinstance(pred, bool):
                _fire_from(table_hbm if pred else table_sh, b)
            else:
                pl.when(pred)(lambda: _fire_from(table_hbm, b))
                pl.when(~pred)(lambda: _fire_from(table_sh, b))

        def wait_gathers(b):
            for j in range(IDX_ROWS):
                pltpu.make_async_copy(
                    table_sh.at[idx_v.at[b, j]],
                    rows_v.at[b, pl.ds(j * GATHER, GATHER)],
                    gsem,
                ).wait()

        def start_out(c, b):
            pltpu.async_copy(rows_v.at[b], out_hbm.at[out_rows_of(c)], osems[b])

        def wait_out(c, b):
            pltpu.make_async_copy(rows_v.at[b], out_hbm.at[out_rows_of(c)], osems[b]).wait()

        def start_idx(c, b):
            pltpu.async_copy(idx_hbm.at[idx_rows_of(c)], idx_v.at[b], isem)

        def wait_idx(c, b):
            pltpu.make_async_copy(idx_hbm.at[idx_rows_of(c)], idx_v.at[b], isem).wait()

        @pl.when(lax.axis_index("s") == 0)
        def _():
            pltpu.sync_copy(table_hbm, table_sh)

        plsc.subcore_barrier()

        pltpu.sync_copy(idx_hbm.at[idx_rows_of(0)], idx_v.at[0])
        fire_gathers(0, 0)
        start_idx(1, 1 % NBUF)

        def step(c, b, traced):
            cond = pl.when if traced else (lambda p: (lambda f: f() if p else None))
            b1, b2 = (b + 1) % NBUF, (b + 2) % NBUF
            wait_gathers(b)
            start_out(c, b)

            @cond(c + 1 < n_chunks)
            def _():
                wait_idx(c + 1, b1)

                @cond(c + 1 >= NBUF)
                def _():
                    wait_out(c + 1 - NBUF, b1)

                fire_gathers(b1, c + 1)

            @cond(c + 2 < n_chunks)
            def _():
                start_idx(c + 2, b2)

        def outer(h, carry):
            for k in range(NBUF):
                step(NBUF * h + k, k, traced=True)
            return carry

        lax.fori_loop(0, n_outer, outer, 0)
        for c, b in tail:
            step(c, b, traced=False)
        for c in range(n_chunks - NBUF, n_chunks):
            wait_out(c, c % NBUF)

    return gather_kernel


def kernel(x, embeddings):
    b, t = x.shape
    total = b * t
    idx2d = x.astype(jnp.int32).reshape(total // GATHER, GATHER)
    out = _build(total)(embeddings, idx2d)
    return out.reshape(b, t, embeddings.shape[1])


# speedup vs baseline: 1.0460x; 1.0460x over previous
import functools

import jax
import jax.numpy as jnp
from jax import lax
from jax.experimental import pallas as pl
from jax.experimental.pallas import tpu as pltpu
from jax.experimental.pallas import tpu_sc as plsc

D = 128
NW = 32
GATHER = 128
IDX_ROWS = 1
CHUNK = IDX_ROWS * GATHER
NBUF = 3
HBM_EVERY = 16
TABLE_ROWS = 8192


def _build(total_rows: int):
    per_w = total_rows // NW
    n_chunks = per_w // CHUNK
    n_outer = n_chunks // NBUF
    tail = [(c, c % NBUF) for c in range(n_outer * NBUF, n_chunks)]
    mesh = plsc.VectorSubcoreMesh(core_axis_name="c", subcore_axis_name="s")

    @functools.partial(
        pl.kernel,
        mesh=mesh,
        out_type=jax.ShapeDtypeStruct((total_rows, D), jnp.float32),
        scratch_types=[
            pltpu.VMEM((NBUF, IDX_ROWS, GATHER), jnp.int32),
            pltpu.VMEM((NBUF, CHUNK, D), jnp.float32),
            pltpu.VMEM_SHARED((TABLE_ROWS, D), jnp.float32),
            pltpu.SemaphoreType.DMA,
            pltpu.SemaphoreType.DMA,
        ] + [pltpu.SemaphoreType.DMA] * NBUF,
    )
    def gather_kernel(table_hbm, idx_hbm, out_hbm, idx_v, rows_v, table_sh,
                      isem, gsem, *osems):
        wid = lax.axis_index("s") * 2 + lax.axis_index("c")
        base_irow = wid * (per_w // GATHER)

        def idx_rows_of(c):
            return pl.ds(base_irow + c * IDX_ROWS, IDX_ROWS)

        def out_rows_of(c):
            return pl.ds((base_irow + c * IDX_ROWS) * GATHER, CHUNK)

        def _fire_from(table_ref, b):
            for j in range(IDX_ROWS):
                pltpu.async_copy(
                    table_ref.at[idx_v.at[b, j]],
                    rows_v.at[b, pl.ds(j * GATHER, GATHER)],
                    gsem,
                )

        def fire_gathers(b, c):
            pred = c % HBM_EVERY == 0
            if isinstance(pred, bool):
                _fire_from(table_hbm if pred else table_sh, b)
            else:
                pl.when(pred)(lambda: _fire_from(table_hbm, b))
                pl.when(~pred)(lambda: _fire_from(table_sh, b))

        def wait_gathers(b):
            for j in range(IDX_ROWS):
                pltpu.make_async_copy(
                    table_sh.at[idx_v.at[b, j]],
                    rows_v.at[b, pl.ds(j * GATHER, GATHER)],
                    gsem,
                ).wait()

        def start_out(c, b):
            pltpu.async_copy(rows_v.at[b], out_hbm.at[out_rows_of(c)], osems[b])

        def wait_out(c, b):
            pltpu.make_async_copy(rows_v.at[b], out_hbm.at[out_rows_of(c)], osems[b]).wait()

        def start_idx(c, b):
            pltpu.async_copy(idx_hbm.at[idx_rows_of(c)], idx_v.at[b], isem)

        def wait_idx(c, b):
            pltpu.make_async_copy(idx_hbm.at[idx_rows_of(c)], idx_v.at[b], isem).wait()

        @pl.when(lax.axis_index("s") == 0)
        def _():
            pltpu.sync_copy(table_hbm, table_sh)

        plsc.subcore_barrier()

        pltpu.sync_copy(idx_hbm.at[idx_rows_of(0)], idx_v.at[0])
        fire_gathers(0, 0)
        start_idx(1, 1 % NBUF)

        def step(c, b, traced):
            cond = pl.when if traced else (lambda p: (lambda f: f() if p else None))
            b1, b2 = (b + 1) % NBUF, (b + 2) % NBUF
            wait_gathers(b)
            start_out(c, b)

            @cond(c + 1 < n_chunks)
            def _():
                wait_idx(c + 1, b1)

                @cond(c + 1 >= NBUF)
                def _():
                    wait_out(c + 1 - NBUF, b1)

                fire_gathers(b1, c + 1)

            @cond(c + 2 < n_chunks)
            def _():
                start_idx(c + 2, b2)

        def outer(h, carry):
            for k in range(NBUF):
                step(NBUF * h + k, k, traced=True)
            return carry

        lax.fori_loop(0, n_outer, outer, 0)
        for c, b in tail:
            step(c, b, traced=False)
        for c in range(n_chunks - NBUF, n_chunks):
            wait_out(c, c % NBUF)

    return gather_kernel


def kernel(x, embeddings):
    b, t = x.shape
    total = b * t
    idx2d = x.astype(jnp.int32).reshape(total // GATHER, GATHER)
    out = _build(total)(embeddings, idx2d)
    return out.reshape(b, t, embeddings.shape[1])


# speedup vs baseline: 1.0786x; 1.0312x over previous
import functools

import jax
import jax.numpy as jnp
from jax import lax
from jax.experimental import pallas as pl
from jax.experimental.pallas import tpu as pltpu
from jax.experimental.pallas import tpu_sc as plsc

D = 128
NW = 32
GATHER = 64
IDX_ROWS = 2
CHUNK = IDX_ROWS * GATHER
NBUF = 3
TABLE_ROWS = 8192


def _build(total_rows: int):
    per_w = total_rows // NW
    n_chunks = per_w // CHUNK
    n_outer = n_chunks // NBUF
    tail = [(c, c % NBUF) for c in range(n_outer * NBUF, n_chunks)]
    mesh = plsc.VectorSubcoreMesh(core_axis_name="c", subcore_axis_name="s")

    @functools.partial(
        pl.kernel,
        mesh=mesh,
        out_type=jax.ShapeDtypeStruct((total_rows, D), jnp.float32),
        scratch_types=[
            pltpu.VMEM((NBUF, IDX_ROWS, GATHER), jnp.int32),
            pltpu.VMEM((NBUF, CHUNK, D), jnp.float32),
            pltpu.VMEM_SHARED((TABLE_ROWS, D), jnp.float32),
            pltpu.SemaphoreType.DMA,
            pltpu.SemaphoreType.DMA,
        ] + [pltpu.SemaphoreType.DMA] * NBUF,
    )
    def gather_kernel(table_hbm, idx_hbm, out_hbm, idx_v, rows_v, table_sh,
                      isem, gsem, *osems):
        wid = lax.axis_index("s") * 2 + lax.axis_index("c")
        base_irow = wid * (per_w // GATHER)

        def idx_rows_of(c):
            return pl.ds(base_irow + c * IDX_ROWS, IDX_ROWS)

        def out_rows_of(c):
            return pl.ds((base_irow + c * IDX_ROWS) * GATHER, CHUNK)

        def _fire_from(table_ref, b):
            for j in range(IDX_ROWS):
                pltpu.async_copy(
                    table_ref.at[idx_v.at[b, j]],
                    rows_v.at[b, pl.ds(j * GATHER, GATHER)],
                    gsem,
                )

        def fire_gathers(b, c):
            _fire_from(table_sh, b)

        def wait_gathers(b):
            for j in range(IDX_ROWS):
                pltpu.make_async_copy(
                    table_sh.at[idx_v.at[b, j]],
                    rows_v.at[b, pl.ds(j * GATHER, GATHER)],
                    gsem,
                ).wait()

        def start_out(c, b):
            pltpu.async_copy(rows_v.at[b], out_hbm.at[out_rows_of(c)], osems[b])

        def wait_out(c, b):
            pltpu.make_async_copy(rows_v.at[b], out_hbm.at[out_rows_of(c)], osems[b]).wait()

        def start_idx(c, b):
            pltpu.async_copy(idx_hbm.at[idx_rows_of(c)], idx_v.at[b], isem)

        def wait_idx(c, b):
            pltpu.make_async_copy(idx_hbm.at[idx_rows_of(c)], idx_v.at[b], isem).wait()

        @pl.when(lax.axis_index("s") == 0)
        def _():
            pltpu.sync_copy(table_hbm, table_sh)

        plsc.subcore_barrier()

        pltpu.sync_copy(idx_hbm.at[idx_rows_of(0)], idx_v.at[0])
        fire_gathers(0, 0)
        start_idx(1, 1 % NBUF)

        def step(c, b, traced):
            cond = pl.when if traced else (lambda p: (lambda f: f() if p else None))
            b1, b2 = (b + 1) % NBUF, (b + 2) % NBUF
            wait_gathers(b)
            start_out(c, b)

            @cond(c + 1 < n_chunks)
            def _():
                wait_idx(c + 1, b1)

                @cond(c + 1 >= NBUF)
                def _():
                    wait_out(c + 1 - NBUF, b1)

                fire_gathers(b1, c + 1)

            @cond(c + 2 < n_chunks)
            def _():
                start_idx(c + 2, b2)

        def outer(h, carry):
            for k in range(NBUF):
                step(NBUF * h + k, k, traced=True)
            return carry

        lax.fori_loop(0, n_outer, outer, 0)
        for c, b in tail:
            step(c, b, traced=False)
        for c in range(n_chunks - NBUF, n_chunks):
            wait_out(c, c % NBUF)

    return gather_kernel


def kernel(x, embeddings):
    b, t = x.shape
    total = b * t
    idx2d = x.astype(jnp.int32).reshape(total // GATHER, GATHER)
    out = _build(total)(embeddings, idx2d)
    return out.reshape(b, t, embeddings.shape[1])


# speedup vs baseline: 1.0958x; 1.0160x over previous
import functools

import jax
import jax.numpy as jnp
from jax import lax
from jax.experimental import pallas as pl
from jax.experimental.pallas import tpu as pltpu
from jax.experimental.pallas import tpu_sc as plsc

D = 128
NW = 32
GATHER = 128
IDX_ROWS = 1
CHUNK = IDX_ROWS * GATHER
NBUF = 3
TABLE_ROWS = 8192


def _build(total_rows: int):
    per_w = total_rows // NW
    n_chunks = per_w // CHUNK
    n_outer = n_chunks // NBUF
    tail = [(c, c % NBUF) for c in range(n_outer * NBUF, n_chunks)]
    mesh = plsc.VectorSubcoreMesh(core_axis_name="c", subcore_axis_name="s")

    @functools.partial(
        pl.kernel,
        mesh=mesh,
        out_type=jax.ShapeDtypeStruct((total_rows, D), jnp.float32),
        scratch_types=[
            pltpu.VMEM((NBUF, IDX_ROWS, GATHER), jnp.int32),
            pltpu.VMEM((NBUF, CHUNK, D), jnp.float32),
            pltpu.VMEM_SHARED((TABLE_ROWS, D), jnp.float32),
            pltpu.SemaphoreType.DMA,
            pltpu.SemaphoreType.DMA,
        ] + [pltpu.SemaphoreType.DMA] * NBUF,
    )
    def gather_kernel(table_hbm, idx_hbm, out_hbm, idx_v, rows_v, table_sh,
                      isem, gsem, *osems):
        wid = lax.axis_index("s") * 2 + lax.axis_index("c")
        base_irow = wid * (per_w // GATHER)

        def idx_rows_of(c):
            return pl.ds(base_irow + c * IDX_ROWS, IDX_ROWS)

        def out_rows_of(c):
            return pl.ds((base_irow + c * IDX_ROWS) * GATHER, CHUNK)

        def _fire_from(table_ref, b):
            for j in range(IDX_ROWS):
                pltpu.async_copy(
                    table_ref.at[idx_v.at[b, j]],
                    rows_v.at[b, pl.ds(j * GATHER, GATHER)],
                    gsem,
                )

        def fire_gathers(b, c):
            _fire_from(table_sh, b)

        def wait_gathers(b):
            for j in range(IDX_ROWS):
                pltpu.make_async_copy(
                    table_sh.at[idx_v.at[b, j]],
                    rows_v.at[b, pl.ds(j * GATHER, GATHER)],
                    gsem,
                ).wait()

        def start_out(c, b):
            pltpu.async_copy(rows_v.at[b], out_hbm.at[out_rows_of(c)], osems[b])

        def wait_out(c, b):
            pltpu.make_async_copy(rows_v.at[b], out_hbm.at[out_rows_of(c)], osems[b]).wait()

        def start_idx(c, b):
            pltpu.async_copy(idx_hbm.at[idx_rows_of(c)], idx_v.at[b], isem)

        def wait_idx(c, b):
            pltpu.make_async_copy(idx_hbm.at[idx_rows_of(c)], idx_v.at[b], isem).wait()

        @pl.when(lax.axis_index("s") == 0)
        def _():
            pltpu.sync_copy(table_hbm, table_sh)

        plsc.subcore_barrier()

        pltpu.sync_copy(idx_hbm.at[idx_rows_of(0)], idx_v.at[0])
        fire_gathers(0, 0)
        start_idx(1, 1 % NBUF)

        def step(c, b, traced):
            cond = pl.when if traced else (lambda p: (lambda f: f() if p else None))
            b1, b2 = (b + 1) % NBUF, (b + 2) % NBUF
            wait_gathers(b)
            start_out(c, b)

            @cond(c + 1 < n_chunks)
            def _():
                wait_idx(c + 1, b1)

                @cond(c + 1 >= NBUF)
                def _():
                    wait_out(c + 1 - NBUF, b1)

                fire_gathers(b1, c + 1)

            @cond(c + 2 < n_chunks)
            def _():
                start_idx(c + 2, b2)

        def outer(h, carry):
            for k in range(NBUF):
                step(NBUF * h + k, k, traced=True)
            return carry

        lax.fori_loop(0, n_outer, outer, 0)
        for c, b in tail:
            step(c, b, traced=False)
        for c in range(n_chunks - NBUF, n_chunks):
            wait_out(c, c % NBUF)

    return gather_kernel


def kernel(x, embeddings):
    b, t = x.shape
    total = b * t
    idx2d = x.astype(jnp.int32).reshape(total // GATHER, GATHER)
    out = _build(total)(embeddings, idx2d)
    return out.reshape(b, t, embeddings.shape[1])


# speedup vs baseline: 1.1757x; 1.0728x over previous
import functools

import jax
import jax.numpy as jnp
from jax import lax
from jax.experimental import pallas as pl
from jax.experimental.pallas import tpu as pltpu
from jax.experimental.pallas import tpu_sc as plsc

D = 128
NW = 32
GATHER = 128
IDX_ROWS = 1
CHUNK = IDX_ROWS * GATHER
NBUF = 3
TABLE_ROWS = 8192


def _build(total_rows: int):
    per_w = total_rows // NW
    n_chunks = per_w // CHUNK
    n_outer = n_chunks // NBUF
    tail = [(c, c % NBUF) for c in range(n_outer * NBUF, n_chunks)]
    mesh = plsc.VectorSubcoreMesh(core_axis_name="c", subcore_axis_name="s")

    @functools.partial(
        pl.kernel,
        mesh=mesh,
        out_type=jax.ShapeDtypeStruct((total_rows, D), jnp.float32),
        scratch_types=[
            pltpu.VMEM((NBUF, IDX_ROWS, GATHER), jnp.int32),
            pltpu.VMEM((NBUF, CHUNK, D), jnp.float32),
            pltpu.VMEM_SHARED((TABLE_ROWS, D), jnp.float32),
            pltpu.SemaphoreType.DMA,
            pltpu.SemaphoreType.DMA,
        ] + [pltpu.SemaphoreType.DMA] * NBUF,
    )
    def gather_kernel(table_hbm, idx_hbm, out_hbm, idx_v, rows_v, table_sh,
                      isem, gsem, *osems):
        wid = lax.axis_index("s") * 2 + lax.axis_index("c")
        base_irow = wid * (per_w // GATHER)

        def idx_rows_of(c):
            return pl.ds(base_irow + c * IDX_ROWS, IDX_ROWS)

        def out_rows_of(c):
            return pl.ds((base_irow + c * IDX_ROWS) * GATHER, CHUNK)

        def _fire_from(table_ref, b):
            for j in range(IDX_ROWS):
                pltpu.async_copy(
                    table_ref.at[idx_v.at[b, j]],
                    rows_v.at[b, pl.ds(j * GATHER, GATHER)],
                    gsem,
                )

        def fire_gathers(b, c):
            _fire_from(table_sh, b)

        def wait_gathers(b):
            for j in range(IDX_ROWS):
                pltpu.make_async_copy(
                    table_sh.at[idx_v.at[b, j]],
                    rows_v.at[b, pl.ds(j * GATHER, GATHER)],
                    gsem,
                ).wait()

        def start_out(c, b):
            pltpu.async_copy(rows_v.at[b], out_hbm.at[out_rows_of(c)], osems[b])

        def wait_out(c, b):
            pltpu.make_async_copy(rows_v.at[b], out_hbm.at[out_rows_of(c)], osems[b]).wait()

        def start_idx(c, b):
            pltpu.async_copy(idx_hbm.at[idx_rows_of(c)], idx_v.at[b], isem)

        def wait_idx(c, b):
            pltpu.make_async_copy(idx_hbm.at[idx_rows_of(c)], idx_v.at[b], isem).wait()

        @pl.when(lax.axis_index("s") == 0)
        def _():
            pltpu.sync_copy(table_hbm, table_sh)

        plsc.subcore_barrier()

        pltpu.sync_copy(idx_hbm.at[idx_rows_of(0)], idx_v.at[0])
        fire_gathers(0, 0)
        pltpu.sync_copy(idx_hbm.at[idx_rows_of(1)], idx_v.at[1])
        fire_gathers(1, 1)
        start_idx(2, 2 % NBUF)

        def step(c, b, traced):
            cond = pl.when if traced else (lambda p: (lambda f: f() if p else None))
            b2 = (b + 2) % NBUF
            wait_gathers(b)
            start_out(c, b)

            @cond(c + 2 < n_chunks)
            def _():
                wait_idx(c + 2, b2)

                @cond(c >= 1)
                def _():
                    wait_out(c - 1, b2)

                fire_gathers(b2, c + 2)

            @cond(c + 3 < n_chunks)
            def _():
                start_idx(c + 3, b)

        def outer(h, carry):
            for k in range(NBUF):
                step(NBUF * h + k, k, traced=True)
            return carry

        lax.fori_loop(0, n_outer, outer, 0)
        for c, b in tail:
            step(c, b, traced=False)
        for c in range(n_chunks - NBUF, n_chunks):
            wait_out(c, c % NBUF)

    return gather_kernel


def kernel(x, embeddings):
    b, t = x.shape
    total = b * t
    idx2d = x.astype(jnp.int32).reshape(total // GATHER, GATHER)
    out = _build(total)(embeddings, idx2d)
    return out.reshape(b, t, embeddings.shape[1])
